# baseline (speedup 1.0000x reference)
import functools

import jax
import jax.numpy as jnp
from jax import lax
from jax.experimental import pallas as pl
from jax.experimental.pallas import tpu as pltpu
from jax.experimental.pallas import tpu_sc as plsc

NUM_EMBEDDINGS = 1000000
EMBEDDING_DIM = 16
BATCH = 16384

LANES = 16
SUPER = 128 // EMBEDDING_DIM
NUM_CORES = 2
NUM_SUBCORES = 16
NUM_WORKERS = NUM_CORES * NUM_SUBCORES
B_PER_W = BATCH // NUM_WORKERS
GROUPS = B_PER_W // LANES
OUT_ROWS_W = B_PER_W * EMBEDDING_DIM // 128

_MESH = plsc.VectorSubcoreMesh(core_axis_name="c", subcore_axis_name="s")


@functools.partial(
    pl.kernel,
    mesh=_MESH,
    out_type=jax.ShapeDtypeStruct((BATCH * EMBEDDING_DIM // 128, 128), jnp.float32),
    scratch_types=[
        pltpu.VMEM((B_PER_W,), jnp.int32),
        pltpu.VMEM((B_PER_W,), jnp.int32),
        pltpu.VMEM((B_PER_W, 128), jnp.float32),
        pltpu.VMEM((OUT_ROWS_W, 128), jnp.float32),
        pltpu.SemaphoreType.DMA,
    ],
    compiler_params=pltpu.CompilerParams(needs_layout_passes=False),
)
def _sc_gather(idx_hbm, table_hbm, out_hbm, idx_v, sidx_v, rows_v, out_v, sem):
    wid = lax.axis_index("s") * NUM_CORES + lax.axis_index("c")
    base = wid * B_PER_W
    pltpu.sync_copy(idx_hbm.at[pl.ds(base, B_PER_W)], idx_v)

    iota = lax.iota(jnp.int32, LANES)

    def shift_body(g, _):
        off = pl.multiple_of(g * LANES, LANES)
        sidx_v[pl.ds(off, LANES)] = idx_v[pl.ds(off, LANES)] >> 3
        return _

    lax.fori_loop(0, GROUPS, shift_body, 0, unroll=4)

    pltpu.async_copy(table_hbm.at[sidx_v], rows_v, sem).wait()

    def extract_body(g, _):
        off = pl.multiple_of(g * LANES, LANES)
        idxg = idx_v[pl.ds(off, LANES)]
        colg = (idxg & 7) << 4
        vrow = g * LANES + iota
        drow = vrow >> 3
        dcol = (vrow & 7) << 4
        for l in range(EMBEDDING_DIM):
            vals = plsc.load_gather(rows_v, [vrow, colg + l])
            plsc.store_scatter(out_v, [drow, dcol + l], vals)
        return _

    lax.fori_loop(0, GROUPS, extract_body, 0)

    pltpu.sync_copy(out_v, out_hbm.at[pl.ds(wid * OUT_ROWS_W, OUT_ROWS_W)])


def kernel(sequence_name, table):
    table2 = table.reshape(NUM_EMBEDDINGS * EMBEDDING_DIM // 128, 128)
    out2 = _sc_gather(sequence_name.astype(jnp.int32), table2)
    return out2.reshape(BATCH, EMBEDDING_DIM)


# speedup vs baseline: 4.9336x; 4.9336x over previous
import functools

import jax
import jax.numpy as jnp
from jax import lax
from jax.experimental import pallas as pl
from jax.experimental.pallas import tpu as pltpu
from jax.experimental.pallas import tpu_sc as plsc

NUM_EMBEDDINGS = 1000000
EMBEDDING_DIM = 16
BATCH = 16384

LANES = 16
NUM_CORES = 2
NUM_SUBCORES = 16
NUM_WORKERS = NUM_CORES * NUM_SUBCORES
B_PER_W = BATCH // NUM_WORKERS
JBATCH = 8
NBATCH = B_PER_W // JBATCH

_MESH = plsc.VectorSubcoreMesh(core_axis_name="c", subcore_axis_name="s")


@functools.partial(
    pl.kernel,
    mesh=_MESH,
    out_type=jax.ShapeDtypeStruct((EMBEDDING_DIM, BATCH), jnp.float32),
    scratch_types=[
        pltpu.VMEM((B_PER_W,), jnp.int32),
        pltpu.VMEM((LANES, EMBEDDING_DIM, 128), jnp.float32),
        pltpu.VMEM((EMBEDDING_DIM, B_PER_W), jnp.float32),
        pltpu.SemaphoreType.DMA,
    ],
    compiler_params=pltpu.CompilerParams(needs_layout_passes=False),
)
def _sc_gather(idx_hbm, table_hbm, out_hbm, idx_v, buf_v, out_v, sem):
    wid = lax.axis_index("s") * NUM_CORES + lax.axis_index("c")
    base = wid * B_PER_W
    pltpu.sync_copy(idx_hbm.at[pl.ds(base, B_PER_W)], idx_v)

    iota = lax.iota(jnp.int32, LANES)

    def body(g, _):
        goff = pl.multiple_of(g * LANES, LANES)
        idxg = idx_v[pl.ds(goff, LANES)]
        copies = []
        for j in range(LANES):
            t = idxg[j]
            cb = pl.multiple_of((t >> 7) << 7, 128)
            for k in range(2):
                copies.append(
                    pltpu.async_copy(
                        table_hbm.at[k, :, pl.ds(cb, 128)],
                        buf_v.at[j, pl.ds(k * 8, 8), :],
                        sem,
                    )
                )
        for cp in copies:
            cp.wait()
        for j in range(LANES):
            t = idxg[j]
            lane = jnp.full((LANES,), t & 127, jnp.int32)
            vals = plsc.load_gather(buf_v.at[j], [iota, lane])
            col = jnp.full((LANES,), g * LANES + j, jnp.int32)
            plsc.store_scatter(out_v, [iota, col], vals)
        return _

    lax.fori_loop(0, B_PER_W // LANES, body, 0)
    pltpu.sync_copy(out_v, out_hbm.at[:, pl.ds(base, B_PER_W)])


def kernel(sequence_name, table):
    table3 = table.T.reshape(2, 8, NUM_EMBEDDINGS)
    out_t = _sc_gather(sequence_name.astype(jnp.int32), table3)
    return out_t.T


# speedup vs baseline: 5.6591x; 1.1471x over previous
import functools

import jax
import jax.numpy as jnp
from jax import lax
from jax.experimental import pallas as pl
from jax.experimental.pallas import tpu as pltpu
from jax.experimental.pallas import tpu_sc as plsc

NUM_EMBEDDINGS = 1000000
EMBEDDING_DIM = 16
BATCH = 16384

LANES = 16
NUM_CORES = 2
NUM_SUBCORES = 16
NUM_WORKERS = NUM_CORES * NUM_SUBCORES
B_PER_W = BATCH // NUM_WORKERS
NGROUPS = B_PER_W // LANES

_MESH = plsc.VectorSubcoreMesh(core_axis_name="c", subcore_axis_name="s")


@functools.partial(
    pl.kernel,
    mesh=_MESH,
    out_type=jax.ShapeDtypeStruct((EMBEDDING_DIM, BATCH), jnp.float32),
    scratch_types=[
        pltpu.VMEM((B_PER_W,), jnp.int32),
        pltpu.VMEM((2, LANES, EMBEDDING_DIM, 128), jnp.float32),
        pltpu.VMEM((EMBEDDING_DIM, B_PER_W), jnp.float32),
        pltpu.SemaphoreType.DMA,
        pltpu.SemaphoreType.DMA,
    ],
    compiler_params=pltpu.CompilerParams(needs_layout_passes=False),
)
def _sc_gather(idx_hbm, table_hbm, out_hbm, idx_v, buf_v, out_v, sem_a, sem_b):
    wid = lax.axis_index("s") * NUM_CORES + lax.axis_index("c")
    base = wid * B_PER_W
    pltpu.sync_copy(idx_hbm.at[pl.ds(base, B_PER_W)], idx_v)

    iota = lax.iota(jnp.int32, LANES)
    sems = (sem_a, sem_b)

    def fire(g, slot):
        idxg = idx_v[pl.ds(pl.multiple_of(g * LANES, LANES), LANES)]
        copies = []
        for j in range(LANES):
            t = idxg[j]
            cb = pl.multiple_of((t >> 7) << 7, 128)
            copies.append(
                pltpu.async_copy(
                    table_hbm.at[:, pl.ds(cb, 128)],
                    buf_v.at[slot, j],
                    sems[slot],
                )
            )
        return idxg, copies

    def extract(g, slot, idxg, copies):
        for cp in copies:
            cp.wait()
        for j in range(LANES):
            t = idxg[j]
            lane = jnp.full((LANES,), t & 127, jnp.int32)
            vals = plsc.load_gather(buf_v.at[slot, j], [iota, lane])
            col = jnp.full((LANES,), g * LANES + j, jnp.int32)
            plsc.store_scatter(out_v, [iota, col], vals)

    CHUNK = 4

    def chunk_body(c, _):
        g0 = c * CHUNK
        pending = fire(g0, 0)
        for u in range(CHUNK):
            nxt = fire(g0 + u + 1, (u + 1) % 2) if u + 1 < CHUNK else None
            extract(g0 + u, u % 2, *pending)
            pending = nxt
        return _

    lax.fori_loop(0, NGROUPS // CHUNK, chunk_body, 0)

    pltpu.sync_copy(out_v, out_hbm.at[:, pl.ds(base, B_PER_W)])


def kernel(sequence_name, table):
    out_t = _sc_gather(sequence_name.astype(jnp.int32), table.T)
    return out_t.T
